# baseline (device time: 126711 ns/iter reference)
import jax
import jax.numpy as jnp
from jax import lax
from jax.experimental import pallas as pl
from jax.experimental.pallas import tpu as pltpu


def _peer(mx):
    return (1 - mx, lax.axis_index("y"), lax.axis_index("z"))


def _peer_barrier(peer):
    bar = pltpu.get_barrier_semaphore()
    pl.semaphore_signal(
        bar, inc=1, device_id=peer, device_id_type=pl.DeviceIdType.MESH
    )
    pl.semaphore_wait(bar, 1)


def _exchange(arrs, collective_id):
    n = len(arrs)

    def body(*refs):
        ins = refs[:n]
        outs = refs[n : 2 * n]
        send_sems, recv_sems = refs[2 * n], refs[2 * n + 1]
        mx = lax.axis_index("x")
        peer = _peer(mx)
        _peer_barrier(peer)
        rdmas = []
        for i in range(n):
            outs[i][pl.ds(mx, 1)] = ins[i][...][None]
            rdma = pltpu.make_async_remote_copy(
                src_ref=ins[i],
                dst_ref=outs[i].at[mx],
                send_sem=send_sems.at[i],
                recv_sem=recv_sems.at[i],
                device_id=peer,
                device_id_type=pl.DeviceIdType.MESH,
            )
            rdma.start()
            rdmas.append(rdma)
        for rdma in rdmas:
            rdma.wait()

    return pl.pallas_call(
        body,
        out_shape=[
            jax.ShapeDtypeStruct((2,) + a.shape, a.dtype) for a in arrs
        ],
        in_specs=[pl.BlockSpec(memory_space=pltpu.VMEM)] * n,
        out_specs=[pl.BlockSpec(memory_space=pltpu.VMEM)] * n,
        scratch_shapes=[
            pltpu.SemaphoreType.DMA((n,)),
            pltpu.SemaphoreType.DMA((n,)),
        ],
        compiler_params=pltpu.CompilerParams(collective_id=collective_id),
    )(*arrs)


def _ffn(xflat, w1, w2, wdense):
    t, d = xflat.shape
    f = w1.shape[2]

    def body(x_ref, w1_ref, w2_ref, wd_ref, out_ref):
        e = pl.program_id(0)
        h = lax.dot_general(
            x_ref[...], w1_ref[0],
            (((1,), (0,)), ((), ())),
            preferred_element_type=jnp.float32,
        )
        h = jnp.maximum(h, 0.0).astype(jnp.bfloat16)
        y = lax.dot_general(
            h, w2_ref[0],
            (((1,), (0,)), ((), ())),
            preferred_element_type=jnp.float32,
        )
        sel = (lax.broadcasted_iota(jnp.int32, (t, 4), 1) == e).astype(
            jnp.float32
        )
        wvec = jnp.sum(wd_ref[...] * sel, axis=1, keepdims=True)
        contrib = y * wvec

        @pl.when(e == 0)
        def _():
            out_ref[...] = contrib

        @pl.when(e > 0)
        def _():
            out_ref[...] = out_ref[...] + contrib

    return pl.pallas_call(
        body,
        grid=(4,),
        out_shape=jax.ShapeDtypeStruct((t, d), jnp.float32),
        in_specs=[
            pl.BlockSpec((t, d), lambda e: (0, 0)),
            pl.BlockSpec((1, d, f), lambda e: (e, 0, 0)),
            pl.BlockSpec((1, f, d), lambda e: (e, 0, 0)),
            pl.BlockSpec((t, 4), lambda e: (0, 0)),
        ],
        out_specs=pl.BlockSpec((t, d), lambda e: (0, 0)),
        compiler_params=pltpu.CompilerParams(
            dimension_semantics=("arbitrary",)
        ),
    )(xflat, w1, w2, wdense)


def _combine(pb, collective_id):
    _, m, d = pb.shape

    def body(pb_ref, out_ref, comm_ref, send_sem, recv_sem):
        mx = lax.axis_index("x")
        peer = _peer(mx)
        _peer_barrier(peer)
        rdma = pltpu.make_async_remote_copy(
            src_ref=pb_ref.at[1 - mx],
            dst_ref=comm_ref,
            send_sem=send_sem,
            recv_sem=recv_sem,
            device_id=peer,
            device_id_type=pl.DeviceIdType.MESH,
        )
        rdma.start()
        rdma.wait()
        mine = pb_ref[pl.ds(mx, 1)].reshape(m, d)
        out_ref[...] = mine.astype(jnp.float32) + comm_ref[...].astype(
            jnp.float32
        )

    return pl.pallas_call(
        body,
        out_shape=jax.ShapeDtypeStruct((m, d), jnp.float32),
        in_specs=[pl.BlockSpec(memory_space=pltpu.VMEM)],
        out_specs=pl.BlockSpec(memory_space=pltpu.VMEM),
        scratch_shapes=[
            pltpu.VMEM((m, d), jnp.bfloat16),
            pltpu.SemaphoreType.DMA,
            pltpu.SemaphoreType.DMA,
        ],
        compiler_params=pltpu.CompilerParams(collective_id=collective_id),
    )(pb)


def kernel(x, router, W1, W2):
    m, d = x.shape
    mx = lax.axis_index("x")

    (r2,) = _exchange([router], collective_id=0)
    router_full = jnp.concatenate([r2[0], r2[1]], axis=1)
    gates = jnp.dot(x, router_full, precision=lax.Precision.HIGHEST)
    topv, topi = lax.top_k(gates, 2)
    w = jax.nn.softmax(topv, axis=-1).astype(jnp.float32)

    xb = x.astype(jnp.bfloat16)
    xf, idxf, wf = _exchange([xb, topi.astype(jnp.int32), w], collective_id=1)

    idx_all = idxf.reshape(2 * m, 2)
    w_all = wf.reshape(2 * m, 2)
    eloc = mx * 4 + jnp.arange(4)
    wdense = jnp.sum(
        w_all[:, :, None] * (idx_all[:, :, None] == eloc[None, None, :]),
        axis=1,
    )

    partial = _ffn(
        xf.reshape(2 * m, d),
        W1.astype(jnp.bfloat16),
        W2.astype(jnp.bfloat16),
        wdense,
    )

    pb = partial.astype(jnp.bfloat16).reshape(2, m, d)
    return _combine(pb, collective_id=2)


# device time: 70654 ns/iter; 1.7934x vs baseline; 1.7934x over previous
import jax
import jax.numpy as jnp
from jax import lax
from jax.experimental import pallas as pl
from jax.experimental.pallas import tpu as pltpu

_CAP_H = 192
_NEG = -3.0e38


def _peer(mx):
    return (1 - mx, lax.axis_index("y"), lax.axis_index("z"))


def _peer_barrier(peer):
    bar = pltpu.get_barrier_semaphore()
    pl.semaphore_signal(
        bar, inc=1, device_id=peer, device_id_type=pl.DeviceIdType.MESH
    )
    pl.semaphore_wait(bar, 1)


def _routing(gt):
    _, m = gt.shape
    row = lax.broadcasted_iota(jnp.int32, (8, m), 0)
    m1 = jnp.max(gt, axis=0, keepdims=True)
    i1 = jnp.min(jnp.where(gt == m1, row, 8), axis=0, keepdims=True)
    g2 = jnp.where(row == i1, _NEG, gt)
    m2 = jnp.max(g2, axis=0, keepdims=True)
    i2 = jnp.min(jnp.where(g2 == m2, row, 8), axis=0, keepdims=True)
    a = jnp.exp(m2 - m1)
    w1 = 1.0 / (1.0 + a)
    w2 = a / (1.0 + a)
    match1 = row == i1
    match2 = row == i2
    assigned = jnp.logical_or(match1, match2)
    wd8 = jnp.where(match1, w1, 0.0) + jnp.where(match2, w2, 0.0)
    tri = (
        lax.broadcasted_iota(jnp.int32, (m, m), 0)
        <= lax.broadcasted_iota(jnp.int32, (m, m), 1)
    ).astype(jnp.float32)
    cum = lax.dot_general(
        assigned.astype(jnp.float32), tri,
        (((1,), (0,)), ((), ())),
        precision=lax.Precision.HIGHEST,
        preferred_element_type=jnp.float32,
    )
    pos8 = jnp.where(assigned, cum.astype(jnp.int32) - 1, -1)
    return pos8, wd8


def _moe(x, router, w1, w2):
    m, d = x.shape
    f = w1.shape[2]

    def body(
        x_ref, r_ref, w1_ref, w2_ref,
        out_ref,
        xsend, xrecv, rbuf, posl, wdl, mpos, mwd, posr, wdr,
        acc0, acc1, psend, prec,
        w1buf, w2buf, w1bb, w2bb,
        send_sems, recv_sems, w1sem, w2sem,
    ):
        e = pl.program_id(0)
        h = pl.program_id(1)
        slot = lax.rem(e, 2)
        mx = lax.axis_index("x")
        peer = _peer(mx)

        def rdma(i, src, dst):
            return pltpu.make_async_remote_copy(
                src_ref=src, dst_ref=dst,
                send_sem=send_sems.at[i], recv_sem=recv_sems.at[i],
                device_id=peer, device_id_type=pl.DeviceIdType.MESH,
            )

        def w_copies(ei, si):
            return [
                pltpu.make_async_copy(w1_ref.at[ei], w1buf.at[si], w1sem.at[si]),
                pltpu.make_async_copy(w2_ref.at[ei], w2buf.at[si], w2sem.at[si]),
            ]

        @pl.when(jnp.logical_and(e == 0, h == 0))
        def _():
            _peer_barrier(peer)
            xsend[...] = x_ref[...].astype(jnp.bfloat16)
            rdma(0, xsend, xrecv).start()
            rdma(1, r_ref, rbuf.at[mx]).start()
            for c in w_copies(0, 0):
                c.start()
            rbuf[pl.ds(mx, 1)] = r_ref[...][None]
            rdma(1, r_ref, rbuf.at[mx]).wait()
            r0 = rbuf[0]
            r1 = rbuf[1]
            rmine = jnp.where(mx == 0, r0, r1)
            rother = jnp.where(mx == 0, r1, r0)
            xv = x_ref[...]
            gmine = lax.dot_general(
                rmine, xv, (((0,), (1,)), ((), ())),
                precision=lax.Precision.HIGHEST,
                preferred_element_type=jnp.float32,
            )
            gother = lax.dot_general(
                rother, xv, (((0,), (1,)), ((), ())),
                precision=lax.Precision.HIGHEST,
                preferred_element_type=jnp.float32,
            )
            pos8, wd8 = _routing(jnp.concatenate([gmine, gother], axis=0))
            posl[...] = pos8[:4][:, None, :]
            wdl[...] = wd8[:4][:, None, :]
            mpos[...] = pos8[4:][:, None, :]
            mwd[...] = wd8[4:][:, None, :]
            rdma(2, mpos, posr).start()
            rdma(3, mwd, wdr).start()

        @pl.when(h == 0)
        def _():
            for c in w_copies(e, slot):
                c.wait()
            w1bb[...] = w1buf[pl.ds(slot, 1)][0].astype(jnp.bfloat16)
            w2bb[...] = w2buf[pl.ds(slot, 1)][0].astype(jnp.bfloat16)

        @pl.when(jnp.logical_and(h == 0, e < 3))
        def _():
            for c in w_copies(e + 1, 1 - slot):
                c.start()

        @pl.when(jnp.logical_and(e == 0, h == 1))
        def _():
            rdma(0, xsend, xrecv).wait()
            rdma(2, mpos, posr).wait()
            rdma(3, mwd, wdr).wait()

        is_loc = jnp.logical_xor(h == 0, e == 3)
        src = jnp.where(is_loc, xsend[...], xrecv[...])
        row4 = lax.broadcasted_iota(jnp.int32, (4, 1, m), 0) == e
        pvec = jnp.sum(jnp.where(is_loc, posl[...], posr[...]) * row4, axis=0)
        wrow = jnp.sum(
            jnp.where(is_loc, wdl[...], wdr[...]) * row4.astype(jnp.float32),
            axis=0,
        )

        w1v = w1bb[...]
        w2v = w2bb[...]

        disp = (
            lax.broadcasted_iota(jnp.int32, (_CAP_H, m), 0) == pvec
        ).astype(jnp.bfloat16)
        xg = lax.dot_general(
            disp, src, (((1,), (0,)), ((), ())),
            preferred_element_type=jnp.float32,
        ).astype(jnp.bfloat16)
        hh = lax.dot_general(
            xg, w1v, (((1,), (0,)), ((), ())),
            preferred_element_type=jnp.float32,
        )
        hh = jnp.maximum(hh, 0.0).astype(jnp.bfloat16)
        y = lax.dot_general(
            hh, w2v, (((1,), (0,)), ((), ())),
            preferred_element_type=jnp.float32,
        ).astype(jnp.bfloat16)
        disp_w = disp * wrow.astype(jnp.bfloat16)
        contrib = lax.dot_general(
            disp_w, y, (((0,), (0,)), ((), ())),
            preferred_element_type=jnp.float32,
        )

        @pl.when(jnp.logical_and(is_loc, e == 0))
        def _():
            acc0[...] = contrib

        @pl.when(jnp.logical_and(is_loc, e > 0))
        def _():
            acc0[...] = acc0[...] + contrib

        @pl.when(jnp.logical_and(jnp.logical_not(is_loc), e == 0))
        def _():
            acc1[...] = contrib

        @pl.when(jnp.logical_and(jnp.logical_not(is_loc), e > 0))
        def _():
            acc1[...] = acc1[...] + contrib

        @pl.when(jnp.logical_and(e == 3, h == 0))
        def _():
            psend[...] = acc1[...].astype(jnp.bfloat16)
            rdma(4, psend, prec).start()

        @pl.when(jnp.logical_and(e == 3, h == 1))
        def _():
            rdma(4, psend, prec).wait()
            out_ref[...] = acc0[...] + prec[...].astype(jnp.float32)

    return pl.pallas_call(
        body,
        grid=(4, 2),
        out_shape=jax.ShapeDtypeStruct((m, d), jnp.float32),
        in_specs=[
            pl.BlockSpec(memory_space=pltpu.VMEM),
            pl.BlockSpec(memory_space=pltpu.VMEM),
            pl.BlockSpec(memory_space=pltpu.MemorySpace.HBM),
            pl.BlockSpec(memory_space=pltpu.MemorySpace.HBM),
        ],
        out_specs=pl.BlockSpec((m, d), lambda e, h: (0, 0)),
        scratch_shapes=[
            pltpu.VMEM((m, d), jnp.bfloat16),
            pltpu.VMEM((m, d), jnp.bfloat16),
            pltpu.VMEM((2, d, 4), jnp.float32),
            pltpu.VMEM((4, 1, m), jnp.int32),
            pltpu.VMEM((4, 1, m), jnp.float32),
            pltpu.VMEM((4, 1, m), jnp.int32),
            pltpu.VMEM((4, 1, m), jnp.float32),
            pltpu.VMEM((4, 1, m), jnp.int32),
            pltpu.VMEM((4, 1, m), jnp.float32),
            pltpu.VMEM((m, d), jnp.float32),
            pltpu.VMEM((m, d), jnp.float32),
            pltpu.VMEM((m, d), jnp.bfloat16),
            pltpu.VMEM((m, d), jnp.bfloat16),
            pltpu.VMEM((2, d, f), jnp.float32),
            pltpu.VMEM((2, f, d), jnp.float32),
            pltpu.VMEM((d, f), jnp.bfloat16),
            pltpu.VMEM((f, d), jnp.bfloat16),
            pltpu.SemaphoreType.DMA((5,)),
            pltpu.SemaphoreType.DMA((5,)),
            pltpu.SemaphoreType.DMA((2,)),
            pltpu.SemaphoreType.DMA((2,)),
        ],
        compiler_params=pltpu.CompilerParams(
            collective_id=0,
            dimension_semantics=("arbitrary", "arbitrary"),
            vmem_limit_bytes=110 * 1024 * 1024,
        ),
    )(x, router, w1, w2)


def kernel(x, router, W1, W2):
    return _moe(x, router, W1, W2)
